# baseline (speedup 1.0000x reference)
import functools

import jax
import jax.numpy as jnp
from jax import lax
from jax.experimental import pallas as pl
from jax.experimental.pallas import tpu as pltpu
from jax.experimental.pallas import tpu_sc as plsc

N_ROWS = 1_000_000
N_COLS = 64
N_OUT_ROWS = 16_384
TOTAL = N_OUT_ROWS * N_COLS

NW = 32
CHUNK = 128
PER_W = TOTAL // NW
N_CHUNKS = PER_W // CHUNK
LAG = 16
LANES = 16


def _gather_body(xf, idxf, outf, idx_v, fidx_v, out_v, sem):
    wid = lax.axis_index("s") * 2 + lax.axis_index("c")
    row0 = wid * N_CHUNKS

    pltpu.sync_copy(idxf.at[pl.ds(row0, N_CHUNKS)], idx_v)

    def compute_fidx(c):
        for w in range(CHUNK // LANES):
            jv = lax.iota(jnp.int32, LANES) + (w % 4) * LANES
            iv = idx_v[c, pl.ds(w * LANES, LANES)]
            fidx_v[c, pl.ds(w * LANES, LANES)] = (iv << 6) + jv

    def fire(c):
        pltpu.make_async_copy(xf.at[fidx_v.at[c]], out_v.at[c], sem).start()

    def drain(c):
        pltpu.make_async_copy(xf.at[fidx_v.at[c]], out_v.at[c], sem).wait()

    def prime(c, _):
        compute_fidx(c)
        fire(c)
        return _

    def steady(c, _):
        compute_fidx(c)
        fire(c)
        drain(c - LAG)
        return _

    def tail(c, _):
        drain(c)
        return _

    lax.fori_loop(0, LAG, prime, None, unroll=False)
    lax.fori_loop(LAG, N_CHUNKS, steady, None, unroll=False)
    lax.fori_loop(N_CHUNKS - LAG, N_CHUNKS, tail, None, unroll=False)

    pltpu.sync_copy(out_v, outf.at[pl.ds(row0, N_CHUNKS)])


@jax.jit
def _gather_sc(xf, idxf):
    mesh = plsc.VectorSubcoreMesh(core_axis_name="c", subcore_axis_name="s")
    return pl.kernel(
        _gather_body,
        out_type=jax.ShapeDtypeStruct((TOTAL // CHUNK, CHUNK), jnp.float32),
        mesh=mesh,
        scratch_types=[
            pltpu.VMEM((N_CHUNKS, CHUNK), jnp.int32),
            pltpu.VMEM((N_CHUNKS, CHUNK), jnp.int32),
            pltpu.VMEM((N_CHUNKS, CHUNK), jnp.float32),
            pltpu.SemaphoreType.DMA,
        ],
    )(xf, idxf)


def kernel(x, dim, index, sparse_grad, out):
    xf = x.reshape(-1)
    idxf = index.astype(jnp.int32).reshape(TOTAL // CHUNK, CHUNK)
    res = _gather_sc(xf, idxf)
    return res.reshape(N_OUT_ROWS, N_COLS)


# speedup vs baseline: 1.0147x; 1.0147x over previous
import jax
import jax.numpy as jnp
from jax import lax
from jax.experimental import pallas as pl
from jax.experimental.pallas import tpu as pltpu
from jax.experimental.pallas import tpu_sc as plsc

N_ROWS = 1_000_000
N_COLS = 64
N_OUT_ROWS = 16_384
TOTAL = N_OUT_ROWS * N_COLS

NW = 32
PER_W = TOTAL // NW
NBLK = 8
BLK = PER_W // NBLK
LANES = 16


def _gather_body(xf, idxf, outf, idx_v, fidx_v, out_v, sem):
    wid = lax.axis_index("s") * 2 + lax.axis_index("c")
    base = wid * PER_W

    pltpu.sync_copy(idxf.at[pl.ds(base, PER_W)], idx_v)

    jvs = [lax.iota(jnp.int32, LANES) + w * LANES for w in range(4)]

    def blk_refs(b):
        return fidx_v.at[pl.ds(b * BLK, BLK)], out_v.at[pl.ds(b * BLK, BLK)]

    def fire(b):
        def step(i, _):
            off = i * N_COLS
            for w in range(4):
                iv = idx_v[pl.ds(off + w * LANES, LANES)]
                fidx_v[pl.ds(off + w * LANES, LANES)] = (iv << 6) | jvs[w]
            return _

        lax.fori_loop(b * (BLK // N_COLS), (b + 1) * (BLK // N_COLS), step,
                      None, unroll=4)
        fref, oref = blk_refs(b)
        pltpu.make_async_copy(xf.at[fref], oref, sem).start()

    for b in range(NBLK):
        fire(b)
    for b in range(NBLK):
        fref, oref = blk_refs(b)
        pltpu.make_async_copy(xf.at[fref], oref, sem).wait()

    pltpu.sync_copy(out_v, outf.at[pl.ds(base, PER_W)])


@jax.jit
def _gather_sc(xf, idxf):
    mesh = plsc.VectorSubcoreMesh(core_axis_name="c", subcore_axis_name="s")
    return pl.kernel(
        _gather_body,
        out_type=jax.ShapeDtypeStruct((TOTAL,), jnp.float32),
        mesh=mesh,
        scratch_types=[
            pltpu.VMEM((PER_W,), jnp.int32),
            pltpu.VMEM((PER_W,), jnp.int32),
            pltpu.VMEM((PER_W,), jnp.float32),
            pltpu.SemaphoreType.DMA,
        ],
    )(xf, idxf)


def kernel(x, dim, index, sparse_grad, out):
    xf = x.reshape(-1)
    idxf = index.astype(jnp.int32).reshape(-1)
    res = _gather_sc(xf, idxf)
    return res.reshape(N_OUT_ROWS, N_COLS)


# speedup vs baseline: 4.0166x; 3.9584x over previous
import jax
import jax.numpy as jnp
from jax import lax
from jax.experimental import pallas as pl
from jax.experimental.pallas import tpu as pltpu
from jax.experimental.pallas import tpu_sc as plsc

N_ROWS = 1_000_000
N_COLS = 64
N_OUT = 16_384

ALIGNED = 999_936
COLS_PER_SC = N_COLS // 2
SEG = N_OUT // 16
SLC = 488 * 128
SLC_LAST = ALIGNED - 15 * SLC


def _gather_body(xt, xtail, idxt, ot, col_a, col_b, idx_v, out_v,
                 sem_a, sem_b, isem_a, isem_b, gsem, osem_a, osem_b):
    cid = lax.axis_index("c")
    sid = lax.axis_index("s")
    j0 = cid * COLS_PER_SC

    def stage_start(col_ref, j, sem):
        @pl.when(sid < 15)
        def _():
            pltpu.make_async_copy(xt.at[j, pl.ds(sid * SLC, SLC)],
                                  col_ref.at[pl.ds(sid * SLC, SLC)], sem).start()

        @pl.when(sid == 15)
        def _():
            pltpu.make_async_copy(xt.at[j, pl.ds(15 * SLC, SLC_LAST)],
                                  col_ref.at[pl.ds(15 * SLC, SLC_LAST)], sem).start()
            pltpu.make_async_copy(xtail.at[pl.ds(j * 128, 128)],
                                  col_ref.at[pl.ds(ALIGNED, 128)], sem).start()

    def stage_wait(col_ref, j, sem):
        @pl.when(sid < 15)
        def _():
            pltpu.make_async_copy(xt.at[j, pl.ds(sid * SLC, SLC)],
                                  col_ref.at[pl.ds(sid * SLC, SLC)], sem).wait()

        @pl.when(sid == 15)
        def _():
            pltpu.make_async_copy(xt.at[j, pl.ds(15 * SLC, SLC_LAST)],
                                  col_ref.at[pl.ds(15 * SLC, SLC_LAST)], sem).wait()
            pltpu.make_async_copy(xtail.at[pl.ds(j * 128, 128)],
                                  col_ref.at[pl.ds(ALIGNED, 128)], sem).wait()

    def idx_slot(k):
        return idx_v.at[pl.ds((k % 2) * SEG, SEG)]

    def out_slot(k):
        return out_v.at[pl.ds((k % 2) * SEG, SEG)]

    def idx_start(k):
        pltpu.make_async_copy(idxt.at[j0 + k, pl.ds(sid * SEG, SEG)],
                              idx_slot(k), isem_a if k % 2 == 0 else isem_b).start()

    def idx_wait(k):
        pltpu.make_async_copy(idxt.at[j0 + k, pl.ds(sid * SEG, SEG)],
                              idx_slot(k), isem_a if k % 2 == 0 else isem_b).wait()

    def out_start(k):
        pltpu.make_async_copy(out_slot(k), ot.at[j0 + k, pl.ds(sid * SEG, SEG)],
                              osem_a if k % 2 == 0 else osem_b).start()

    def out_wait(k):
        pltpu.make_async_copy(out_slot(k), ot.at[j0 + k, pl.ds(sid * SEG, SEG)],
                              osem_a if k % 2 == 0 else osem_b).wait()

    stage_start(col_a, j0, sem_a)
    idx_start(0)

    for k in range(COLS_PER_SC):
        j = j0 + k
        buf, sem = (col_a, sem_a) if k % 2 == 0 else (col_b, sem_b)
        if k + 1 < COLS_PER_SC:
            nbuf, nsem = (col_b, sem_b) if k % 2 == 0 else (col_a, sem_a)
            stage_start(nbuf, j + 1, nsem)
            idx_start(k + 1)
        stage_wait(buf, j, sem)
        idx_wait(k)
        if k >= 2:
            out_wait(k - 2)
        plsc.subcore_barrier()

        pltpu.make_async_copy(buf.at[idx_slot(k)], out_slot(k), gsem).start()
        pltpu.make_async_copy(buf.at[idx_slot(k)], out_slot(k), gsem).wait()
        out_start(k)
        plsc.subcore_barrier()

    out_wait(COLS_PER_SC - 2)
    out_wait(COLS_PER_SC - 1)


@jax.jit
def _gather_sc(xt, xtail, idxt):
    mesh = plsc.VectorSubcoreMesh(core_axis_name="c", subcore_axis_name="s")
    return pl.kernel(
        _gather_body,
        out_type=jax.ShapeDtypeStruct((N_COLS, N_OUT), jnp.float32),
        mesh=mesh,
        scratch_types=[
            pltpu.VMEM_SHARED((ALIGNED + 128,), jnp.float32),
            pltpu.VMEM_SHARED((ALIGNED + 128,), jnp.float32),
            pltpu.VMEM((2 * SEG,), jnp.int32),
            pltpu.VMEM((2 * SEG,), jnp.float32),
            pltpu.SemaphoreType.DMA,
            pltpu.SemaphoreType.DMA,
            pltpu.SemaphoreType.DMA,
            pltpu.SemaphoreType.DMA,
            pltpu.SemaphoreType.DMA,
            pltpu.SemaphoreType.DMA,
            pltpu.SemaphoreType.DMA,
        ],
    )(xt, xtail, idxt)


def kernel(x, dim, index, sparse_grad, out):
    xtail = jnp.pad(x[ALIGNED:, :], ((0, 128 - (N_ROWS - ALIGNED)), (0, 0)))
    res_t = _gather_sc(x.T, xtail.T.reshape(-1), index.astype(jnp.int32).T)
    return res_t.T
